# Initial kernel scaffold; baseline (speedup 1.0000x reference)
import jax
import jax.numpy as jnp
from jax.experimental import pallas as pl


def kernel(x, eps0, eps1, eps2, W1, b1, W2, b2, W3, b3, Wa1, Wa2, Wa3, Wc, bc, Wd1, bd1, Wd2, bd2, edge_index, graph_ids):
    raise NotImplementedError("write your pallas kernel here")


# speedup vs baseline: 3.3127x; 3.3127x over previous
import functools

import jax
import jax.numpy as jnp
from jax import lax
from jax.experimental import pallas as pl
from jax.experimental.pallas import tpu as pltpu
from jax.experimental.pallas import tpu_sc as plsc

N = 10000
NP = 10240
E = 320000
EP = 2560 * 128
B = 16
K = 16
H = 256
C = 64
BN = 1280
GRID = NP // BN
RT = 160
NEG = -1e30


@functools.lru_cache(maxsize=None)
def _make_sc_agg(wh):
    mesh = plsc.VectorSubcoreMesh(
        core_axis_name="c", subcore_axis_name="s", num_cores=2, num_subcores=16
    )

    @functools.partial(
        pl.kernel,
        out_type=jax.ShapeDtypeStruct((2 * NP, wh), jnp.float32),
        mesh=mesh,
        scratch_types=[
            pltpu.VMEM_SHARED((NP, wh), jnp.float32),
            pltpu.VMEM((16, 128), jnp.int32),
            pltpu.VMEM((16, 128), jnp.int32),
            pltpu.VMEM((128, wh), jnp.float32),
            pltpu.VMEM((128, wh), jnp.float32),
            pltpu.SemaphoreType.DMA,
            pltpu.SemaphoreType.DMA,
            pltpu.SemaphoreType.DMA,
            pltpu.SemaphoreType.DMA,
        ],
    )
    def sc_agg(feat_hbm, src0_hbm, src1_hbm, dst_hbm, zero_hbm, out_hbm,
               acc, src_c, dst_c, rb0, rb1, sem0, sem1, ssem0, ssem1):
        c = lax.axis_index("c")
        s = lax.axis_index("s")

        pltpu.sync_copy(zero_hbm, rb0)
        for i in range(5):
            pltpu.sync_copy(rb0, acc.at[pl.ds(s * 640 + i * 128, 128)])
        plsc.subcore_barrier()

        tile_row0 = s * RT

        def chunk(ck, carry):
            row0 = tile_row0 + ck * 16

            @pl.when(c == 0)
            def _():
                pltpu.sync_copy(src0_hbm.at[pl.ds(row0, 16)], src_c)

            @pl.when(c == 1)
            def _():
                pltpu.sync_copy(src1_hbm.at[pl.ds(row0, 16)], src_c)

            pltpu.sync_copy(dst_hbm.at[pl.ds(row0, 16)], dst_c)
            for jj in range(8):
                j0 = jj * 2
                cp0 = pltpu.async_copy(feat_hbm.at[src_c.at[j0]], rb0, sem0)
                cp1 = pltpu.async_copy(feat_hbm.at[src_c.at[j0 + 1]], rb1,
                                       sem1)
                cp0.wait()
                pltpu.sync_copy(rb0, acc.at[dst_c.at[j0]], add=True)
                cp1.wait()
                pltpu.sync_copy(rb1, acc.at[dst_c.at[j0 + 1]], add=True)
            return carry

        lax.fori_loop(0, RT // 16, chunk, 0)
        plsc.subcore_barrier()

        for i in range(5):
            r0 = s * 640 + i * 128
            pltpu.sync_copy(acc.at[pl.ds(r0, 128)], rb0)
            pltpu.sync_copy(rb0, out_hbm.at[pl.ds(c * NP + r0, 128)])

    return sc_agg


@functools.lru_cache(maxsize=None)
def _make_sc_agg_edgesplit():
    mesh = plsc.VectorSubcoreMesh(
        core_axis_name="c", subcore_axis_name="s", num_cores=2, num_subcores=16
    )
    rt = 80

    @functools.partial(
        pl.kernel,
        out_type=jax.ShapeDtypeStruct((2 * NP, 128), jnp.float32),
        mesh=mesh,
        scratch_types=[
            pltpu.VMEM_SHARED((NP, 128), jnp.float32),
            pltpu.VMEM((16, 128), jnp.int32),
            pltpu.VMEM((16, 128), jnp.int32),
            pltpu.VMEM((128, 128), jnp.float32),
            pltpu.VMEM((128, 128), jnp.float32),
            pltpu.SemaphoreType.DMA,
            pltpu.SemaphoreType.DMA,
            pltpu.SemaphoreType.DMA,
            pltpu.SemaphoreType.DMA,
        ],
    )
    def sc_agg1(feat_hbm, src_hbm, dst_hbm, zero_hbm, out_hbm,
                acc, src_c, dst_c, rb0, rb1, sem0, sem1, ssem0, ssem1):
        c = lax.axis_index("c")
        s = lax.axis_index("s")

        pltpu.sync_copy(zero_hbm, rb0)
        for i in range(5):
            pltpu.sync_copy(rb0, acc.at[pl.ds(s * 640 + i * 128, 128)])
        plsc.subcore_barrier()

        tile_row0 = (c * 16 + s) * rt

        def chunk(ck, carry):
            row0 = tile_row0 + ck * 16
            pltpu.sync_copy(src_hbm.at[pl.ds(row0, 16)], src_c)
            pltpu.sync_copy(dst_hbm.at[pl.ds(row0, 16)], dst_c)
            for jj in range(8):
                j0 = jj * 2
                cp0 = pltpu.async_copy(feat_hbm.at[src_c.at[j0]], rb0, sem0)
                cp1 = pltpu.async_copy(feat_hbm.at[src_c.at[j0 + 1]], rb1,
                                       sem1)
                cp0.wait()
                pltpu.sync_copy(rb0, acc.at[dst_c.at[j0]], add=True)
                cp1.wait()
                pltpu.sync_copy(rb1, acc.at[dst_c.at[j0 + 1]], add=True)
            return carry

        lax.fori_loop(0, rt // 16, chunk, 0)
        plsc.subcore_barrier()

        for i in range(5):
            r0 = s * 640 + i * 128
            pltpu.sync_copy(acc.at[pl.ds(r0, 128)], rb0)
            pltpu.sync_copy(rb0, out_hbm.at[pl.ds(c * NP + r0, 128)])

    return sc_agg1


def _finish_layer_a(z, wa_ref, gid_ref, featp_ref, scores_ref, smax_ref,
                    smax_acc):
    i = pl.program_id(0)
    fp = jnp.where(z > 0, z, 0.01 * z)
    featp_ref[0] = fp[:, :128]
    featp_ref[1] = fp[:, 128:]
    sc_blk = jnp.dot(fp, wa_ref[...], preferred_element_type=jnp.float32)
    scores_ref[...] = sc_blk

    @pl.when(i == 0)
    def _():
        smax_acc[...] = jnp.full((B, K), NEG, jnp.float32)

    gid = gid_ref[...]
    rows = []
    for b in range(B):
        mb = jnp.where(gid == b, sc_blk, NEG)
        rows.append(jnp.max(mb, axis=0, keepdims=True))
    smax_acc[...] = jnp.maximum(smax_acc[...], jnp.concatenate(rows, axis=0))
    smax_ref[...] = smax_acc[...]


def _tc_layer_a_body(feat_ref, agg_ref, wm_ref, b_ref, wa_ref, s_ref, gid_ref,
                     featp_ref, scores_ref, smax_ref, smax_acc):
    sc = s_ref[0, 0]
    h0 = feat_ref[0] * sc + agg_ref[0]
    h1 = feat_ref[1] * sc + agg_ref[1]
    z = (jnp.dot(h0, wm_ref[0], preferred_element_type=jnp.float32)
         + jnp.dot(h1, wm_ref[1], preferred_element_type=jnp.float32)
         + b_ref[...])
    _finish_layer_a(z, wa_ref, gid_ref, featp_ref, scores_ref, smax_ref,
                    smax_acc)


def _tc_layer_a1_body(feat_ref, agg_ref, w_ref, b_ref, wa_ref, s_ref, gid_ref,
                      featp_ref, scores_ref, smax_ref, smax_acc):
    sc = s_ref[0, 0]
    h = feat_ref[...] * sc + agg_ref[0] + agg_ref[1]
    z = jnp.dot(h, w_ref[...], preferred_element_type=jnp.float32) + b_ref[...]
    _finish_layer_a(z, wa_ref, gid_ref, featp_ref, scores_ref, smax_ref,
                    smax_acc)


def _tc_layer_a1(xp, aggp, w1, bvec, wa, scale, gid2):
    return pl.pallas_call(
        _tc_layer_a1_body,
        grid=(GRID,),
        in_specs=[
            pl.BlockSpec((BN, 128), lambda i: (i, 0)),
            pl.BlockSpec((2, BN, 128), lambda i: (0, i, 0)),
            pl.BlockSpec((128, H), lambda i: (0, 0)),
            pl.BlockSpec((1, H), lambda i: (0, 0)),
            pl.BlockSpec((H, K), lambda i: (0, 0)),
            pl.BlockSpec((1, 1), lambda i: (0, 0)),
            pl.BlockSpec((BN, 1), lambda i: (i, 0)),
        ],
        out_specs=[
            pl.BlockSpec((2, BN, 128), lambda i: (0, i, 0)),
            pl.BlockSpec((BN, K), lambda i: (i, 0)),
            pl.BlockSpec((B, K), lambda i: (0, 0)),
        ],
        out_shape=[
            jax.ShapeDtypeStruct((2, NP, 128), jnp.float32),
            jax.ShapeDtypeStruct((NP, K), jnp.float32),
            jax.ShapeDtypeStruct((B, K), jnp.float32),
        ],
        scratch_shapes=[pltpu.VMEM((B, K), jnp.float32)],
    )(xp, aggp, w1, bvec, wa, scale, gid2)


def _tc_layer_a(feat2, agg2, wm, bvec, wa, scale, gid2):
    whin = feat2.shape[2]
    return pl.pallas_call(
        _tc_layer_a_body,
        grid=(GRID,),
        in_specs=[
            pl.BlockSpec((2, BN, whin), lambda i: (0, i, 0)),
            pl.BlockSpec((2, BN, whin), lambda i: (0, i, 0)),
            pl.BlockSpec((2, whin, H), lambda i: (0, 0, 0)),
            pl.BlockSpec((1, H), lambda i: (0, 0)),
            pl.BlockSpec((H, K), lambda i: (0, 0)),
            pl.BlockSpec((1, 1), lambda i: (0, 0)),
            pl.BlockSpec((BN, 1), lambda i: (i, 0)),
        ],
        out_specs=[
            pl.BlockSpec((2, BN, 128), lambda i: (0, i, 0)),
            pl.BlockSpec((BN, K), lambda i: (i, 0)),
            pl.BlockSpec((B, K), lambda i: (0, 0)),
        ],
        out_shape=[
            jax.ShapeDtypeStruct((2, NP, 128), jnp.float32),
            jax.ShapeDtypeStruct((NP, K), jnp.float32),
            jax.ShapeDtypeStruct((B, K), jnp.float32),
        ],
        scratch_shapes=[pltpu.VMEM((B, K), jnp.float32)],
    )(feat2, agg2, wm, bvec, wa, scale, gid2)


def _tc_layer_b_body(featp_ref, sc_ref, smax_ref, gid_ref,
                     ro_ref, den_ref, ro_acc, den_acc):
    i = pl.program_id(0)

    @pl.when(i == 0)
    def _():
        ro_acc[...] = jnp.zeros((B * K, H), jnp.float32)
        den_acc[...] = jnp.zeros((1, B * K), jnp.float32)

    gid = gid_ref[...]
    iota16 = lax.broadcasted_iota(jnp.int32, (BN, B), 1)
    ohf = (gid == iota16).astype(jnp.float32)
    smaxn = jnp.dot(ohf, smax_ref[...], preferred_element_type=jnp.float32)
    ex = jnp.exp(sc_ref[...] - smaxn)
    ext = jnp.concatenate([ex] * B, axis=1)
    bcol = lax.broadcasted_iota(jnp.int32, (BN, B * K), 1) // K
    a = jnp.where(gid == bcol, ext, 0.0)
    fp = jnp.concatenate([featp_ref[0], featp_ref[1]], axis=1)
    ro_acc[...] += lax.dot_general(
        a, fp, (((0,), (0,)), ((), ())), preferred_element_type=jnp.float32)
    den_acc[...] += jnp.sum(a, axis=0, keepdims=True)
    ro_ref[...] = ro_acc[...]
    den_ref[...] = den_acc[...]


def _tc_layer_b(featp2, scores, smax, gid2):
    return pl.pallas_call(
        _tc_layer_b_body,
        grid=(GRID,),
        in_specs=[
            pl.BlockSpec((2, BN, 128), lambda i: (0, i, 0)),
            pl.BlockSpec((BN, K), lambda i: (i, 0)),
            pl.BlockSpec((B, K), lambda i: (0, 0)),
            pl.BlockSpec((BN, 1), lambda i: (i, 0)),
        ],
        out_specs=[
            pl.BlockSpec((B * K, H), lambda i: (0, 0)),
            pl.BlockSpec((1, B * K), lambda i: (0, 0)),
        ],
        out_shape=[
            jax.ShapeDtypeStruct((B * K, H), jnp.float32),
            jax.ShapeDtypeStruct((1, B * K), jnp.float32),
        ],
        scratch_shapes=[
            pltpu.VMEM((B * K, H), jnp.float32),
            pltpu.VMEM((1, B * K), jnp.float32),
        ],
    )(featp2, scores, smax, gid2)


def _tc_head1_body(ro1, dc1, ro2, dc2, ro3, dc3, wct_ref, bc_ref, out_ref):
    def norm(ro, dc):
        d = dc[...]
        d = jnp.where(d > 0, d, 1.0)
        return ro[...] / d

    m = jnp.concatenate([norm(ro1, dc1), norm(ro2, dc2), norm(ro3, dc3)],
                        axis=1)
    ct = jnp.dot(m, wct_ref[...], preferred_element_type=jnp.float32) + bc_ref[...]
    out_ref[...] = jnp.where(ct > 0, ct, 0.01 * ct)


def _tc_head1(ro1, dc1, ro2, dc2, ro3, dc3, wct, bc2):
    return pl.pallas_call(
        _tc_head1_body,
        out_shape=jax.ShapeDtypeStruct((B * K, C), jnp.float32),
    )(ro1, dc1, ro2, dc2, ro3, dc3, wct, bc2)


def _tc_head2_body(fc_ref, wd1_ref, bd1_ref, wd2_ref, bd2_ref, out_ref):
    d1 = jnp.dot(fc_ref[...], wd1_ref[...], preferred_element_type=jnp.float32)
    d1 = d1 + bd1_ref[...]
    d1 = jnp.where(d1 > 0, d1, 0.01 * d1)
    d2 = jnp.dot(d1, wd2_ref[...], preferred_element_type=jnp.float32)
    out_ref[...] = jax.nn.sigmoid(d2 + bd2_ref[...])


def _tc_head2(fc2, wd1q, bd1, wd2, bd2):
    return pl.pallas_call(
        _tc_head2_body,
        out_shape=jax.ShapeDtypeStruct((B, 2), jnp.float32),
    )(fc2, wd1q, bd1, wd2, bd2)


def kernel(x, eps0, eps1, eps2, W1, b1, W2, b2, W3, b3, Wa1, Wa2, Wa3,
           Wc, bc, Wd1, bd1, Wd2, bd2, edge_index, graph_ids):
    f32 = jnp.float32
    xp = jnp.pad(x, ((0, NP - N), (0, 0)))
    gid2 = jnp.pad(graph_ids, (0, NP - N), constant_values=B).reshape(NP, 1)
    order = jnp.argsort(edge_index[0])
    src = jnp.pad(edge_index[0][order], (0, EP - E)).reshape(2560, 128)
    dst = jnp.pad(edge_index[1][order], (0, EP - E),
                  constant_values=N + 200).reshape(2560, 128)
    src1 = src + NP
    zero128 = jnp.zeros((128, 128), f32)

    wm2 = W2.reshape(2, 128, H)
    wm3 = W3.reshape(2, 128, H)
    s0 = (1.0 + eps0).reshape(1, 1)
    s1 = (1.0 + eps1).reshape(1, 1)
    s2 = (1.0 + eps2).reshape(1, 1)
    wct = Wc.T
    bc2 = bc.reshape(1, C)
    wd1q = Wd1.reshape(C, K, 128).transpose(1, 0, 2).reshape(C * K, 128)
    bd12 = bd1.reshape(1, 128)
    bd22 = bd2.reshape(1, 2)

    agg1 = _make_sc_agg_edgesplit()(xp, src, dst, zero128)
    f1, sc1, smax1 = _tc_layer_a1(xp, agg1.reshape(2, NP, 128), W1,
                                  b1.reshape(1, H), Wa1, s0, gid2)
    ro1, den1 = _tc_layer_b(f1, sc1, smax1, gid2)

    agg2 = _make_sc_agg(128)(f1.reshape(2 * NP, 128), src, src1, dst, zero128)
    f2, sc2, smax2 = _tc_layer_a(f1, agg2.reshape(2, NP, 128), wm2,
                                 b2.reshape(1, H), Wa2, s1, gid2)
    ro2, den2 = _tc_layer_b(f2, sc2, smax2, gid2)

    agg3 = _make_sc_agg(128)(f2.reshape(2 * NP, 128), src, src1, dst, zero128)
    f3, sc3, smax3 = _tc_layer_a(f2, agg3.reshape(2, NP, 128), wm3,
                                 b3.reshape(1, H), Wa3, s2, gid2)
    ro3, den3 = _tc_layer_b(f3, sc3, smax3, gid2)

    conv = _tc_head1(ro1, den1.reshape(B * K, 1), ro2, den2.reshape(B * K, 1),
                     ro3, den3.reshape(B * K, 1), wct, bc2)
    fc2 = conv.reshape(B, C * K)
    return _tc_head2(fc2, wd1q, bd12, Wd2, bd22)


# speedup vs baseline: 4.1488x; 1.2524x over previous
import functools

import jax
import jax.numpy as jnp
from jax import lax
from jax.experimental import pallas as pl
from jax.experimental.pallas import tpu as pltpu
from jax.experimental.pallas import tpu_sc as plsc

N = 10000
NP = 10240
E = 320000
EP = 2560 * 128
B = 16
K = 16
H = 256
C = 64
BN = 1280
GRID = NP // BN
RT = 160
NEG = -1e30


@functools.lru_cache(maxsize=None)
def _make_sc_agg(wh):
    mesh = plsc.VectorSubcoreMesh(
        core_axis_name="c", subcore_axis_name="s", num_cores=2, num_subcores=16
    )

    @functools.partial(
        pl.kernel,
        out_type=jax.ShapeDtypeStruct((2 * NP, wh), jnp.float32),
        mesh=mesh,
        scratch_types=[
            pltpu.VMEM_SHARED((NP, wh), jnp.float32),
            pltpu.VMEM((16, 128), jnp.int32),
            pltpu.VMEM((16, 128), jnp.int32),
            pltpu.VMEM((128, wh), jnp.float32),
            pltpu.VMEM((128, wh), jnp.float32),
            pltpu.SemaphoreType.DMA,
            pltpu.SemaphoreType.DMA,
            pltpu.SemaphoreType.DMA,
            pltpu.SemaphoreType.DMA,
        ],
    )
    def sc_agg(feat_hbm, src0_hbm, src1_hbm, dst_hbm, zero_hbm, out_hbm,
               acc, src_c, dst_c, rb0, rb1, sem0, sem1, ssem0, ssem1):
        c = lax.axis_index("c")
        s = lax.axis_index("s")

        pltpu.sync_copy(zero_hbm, rb0)
        for i in range(5):
            pltpu.sync_copy(rb0, acc.at[pl.ds(s * 640 + i * 128, 128)])
        plsc.subcore_barrier()

        tile_row0 = s * RT

        def chunk(ck, carry):
            row0 = tile_row0 + ck * 16

            @pl.when(c == 0)
            def _():
                pltpu.sync_copy(src0_hbm.at[pl.ds(row0, 16)], src_c)

            @pl.when(c == 1)
            def _():
                pltpu.sync_copy(src1_hbm.at[pl.ds(row0, 16)], src_c)

            pltpu.sync_copy(dst_hbm.at[pl.ds(row0, 16)], dst_c)
            for jj in range(8):
                j0 = jj * 2
                cp0 = pltpu.async_copy(feat_hbm.at[src_c.at[j0]], rb0, sem0)
                cp1 = pltpu.async_copy(feat_hbm.at[src_c.at[j0 + 1]], rb1,
                                       sem1)
                cp0.wait()
                pltpu.sync_copy(rb0, acc.at[dst_c.at[j0]], add=True)
                cp1.wait()
                pltpu.sync_copy(rb1, acc.at[dst_c.at[j0 + 1]], add=True)
            return carry

        lax.fori_loop(0, RT // 16, chunk, 0)
        plsc.subcore_barrier()

        for i in range(5):
            r0 = s * 640 + i * 128
            pltpu.sync_copy(acc.at[pl.ds(r0, 128)], rb0)
            pltpu.sync_copy(rb0, out_hbm.at[pl.ds(c * NP + r0, 128)])

    return sc_agg


@functools.lru_cache(maxsize=None)
def _make_sc_agg_edgesplit():
    mesh = plsc.VectorSubcoreMesh(
        core_axis_name="c", subcore_axis_name="s", num_cores=2, num_subcores=16
    )
    rt = 80

    @functools.partial(
        pl.kernel,
        out_type=jax.ShapeDtypeStruct((2 * NP, 128), jnp.float32),
        mesh=mesh,
        scratch_types=[
            pltpu.VMEM_SHARED((NP, 128), jnp.float32),
            pltpu.VMEM((16, 128), jnp.int32),
            pltpu.VMEM((16, 128), jnp.int32),
            pltpu.VMEM((128, 128), jnp.float32),
            pltpu.VMEM((128, 128), jnp.float32),
            pltpu.SemaphoreType.DMA,
            pltpu.SemaphoreType.DMA,
            pltpu.SemaphoreType.DMA,
            pltpu.SemaphoreType.DMA,
        ],
    )
    def sc_agg1(feat_hbm, src_hbm, dst_hbm, zero_hbm, out_hbm,
                acc, src_c, dst_c, rb0, rb1, sem0, sem1, ssem0, ssem1):
        c = lax.axis_index("c")
        s = lax.axis_index("s")

        pltpu.sync_copy(zero_hbm, rb0)
        for i in range(5):
            pltpu.sync_copy(rb0, acc.at[pl.ds(s * 640 + i * 128, 128)])
        plsc.subcore_barrier()

        tile_row0 = (c * 16 + s) * rt

        def chunk(ck, carry):
            row0 = tile_row0 + ck * 16
            pltpu.sync_copy(src_hbm.at[pl.ds(row0, 16)], src_c)
            pltpu.sync_copy(dst_hbm.at[pl.ds(row0, 16)], dst_c)
            for jj in range(8):
                j0 = jj * 2
                cp0 = pltpu.async_copy(feat_hbm.at[src_c.at[j0]], rb0, sem0)
                cp1 = pltpu.async_copy(feat_hbm.at[src_c.at[j0 + 1]], rb1,
                                       sem1)
                cp0.wait()
                pltpu.sync_copy(rb0, acc.at[dst_c.at[j0]], add=True)
                cp1.wait()
                pltpu.sync_copy(rb1, acc.at[dst_c.at[j0 + 1]], add=True)
            return carry

        lax.fori_loop(0, rt // 16, chunk, 0)
        plsc.subcore_barrier()

        for i in range(5):
            r0 = s * 640 + i * 128
            pltpu.sync_copy(acc.at[pl.ds(r0, 128)], rb0)
            pltpu.sync_copy(rb0, out_hbm.at[pl.ds(c * NP + r0, 128)])

    return sc_agg1


def _finish_layer_a(z, wa_ref, gid_ref, featp_ref, scores_ref, smax_ref,
                    smax_acc):
    i = pl.program_id(0)
    fp = jnp.where(z > 0, z, 0.01 * z)
    featp_ref[0] = fp[:, :128]
    featp_ref[1] = fp[:, 128:]
    sc_blk = jnp.dot(fp, wa_ref[...], preferred_element_type=jnp.float32)
    scores_ref[...] = sc_blk

    @pl.when(i == 0)
    def _():
        smax_acc[...] = jnp.full((B, K), NEG, jnp.float32)

    gid = gid_ref[...]
    rows = []
    for b in range(B):
        mb = jnp.where(gid == b, sc_blk, NEG)
        rows.append(jnp.max(mb, axis=0, keepdims=True))
    smax_acc[...] = jnp.maximum(smax_acc[...], jnp.concatenate(rows, axis=0))
    smax_ref[...] = smax_acc[...]


def _tc_layer_a_body(feat_ref, agg_ref, wm_ref, b_ref, wa_ref, s_ref, gid_ref,
                     featp_ref, scores_ref, smax_ref, smax_acc):
    sc = s_ref[0, 0]
    h0 = feat_ref[0] * sc + agg_ref[0]
    h1 = feat_ref[1] * sc + agg_ref[1]
    z = (jnp.dot(h0, wm_ref[0], preferred_element_type=jnp.float32)
         + jnp.dot(h1, wm_ref[1], preferred_element_type=jnp.float32)
         + b_ref[...])
    _finish_layer_a(z, wa_ref, gid_ref, featp_ref, scores_ref, smax_ref,
                    smax_acc)


def _tc_layer_a1_body(feat_ref, agg_ref, w_ref, b_ref, wa_ref, s_ref, gid_ref,
                      featp_ref, scores_ref, smax_ref, smax_acc):
    sc = s_ref[0, 0]
    h = feat_ref[...] * sc + agg_ref[0] + agg_ref[1]
    z = jnp.dot(h, w_ref[...], preferred_element_type=jnp.float32) + b_ref[...]
    _finish_layer_a(z, wa_ref, gid_ref, featp_ref, scores_ref, smax_ref,
                    smax_acc)


def _tc_layer_a1(xp, aggp, w1, bvec, wa, scale, gid2):
    return pl.pallas_call(
        _tc_layer_a1_body,
        grid=(GRID,),
        in_specs=[
            pl.BlockSpec((BN, 128), lambda i: (i, 0)),
            pl.BlockSpec((2, BN, 128), lambda i: (0, i, 0)),
            pl.BlockSpec((128, H), lambda i: (0, 0)),
            pl.BlockSpec((1, H), lambda i: (0, 0)),
            pl.BlockSpec((H, K), lambda i: (0, 0)),
            pl.BlockSpec((1, 1), lambda i: (0, 0)),
            pl.BlockSpec((BN, 1), lambda i: (i, 0)),
        ],
        out_specs=[
            pl.BlockSpec((2, BN, 128), lambda i: (0, i, 0)),
            pl.BlockSpec((BN, K), lambda i: (i, 0)),
            pl.BlockSpec((B, K), lambda i: (0, 0)),
        ],
        out_shape=[
            jax.ShapeDtypeStruct((2, NP, 128), jnp.float32),
            jax.ShapeDtypeStruct((NP, K), jnp.float32),
            jax.ShapeDtypeStruct((B, K), jnp.float32),
        ],
        scratch_shapes=[pltpu.VMEM((B, K), jnp.float32)],
    )(xp, aggp, w1, bvec, wa, scale, gid2)


def _tc_layer_a(feat2, agg2, wm, bvec, wa, scale, gid2):
    whin = feat2.shape[2]
    return pl.pallas_call(
        _tc_layer_a_body,
        grid=(GRID,),
        in_specs=[
            pl.BlockSpec((2, BN, whin), lambda i: (0, i, 0)),
            pl.BlockSpec((2, BN, whin), lambda i: (0, i, 0)),
            pl.BlockSpec((2, whin, H), lambda i: (0, 0, 0)),
            pl.BlockSpec((1, H), lambda i: (0, 0)),
            pl.BlockSpec((H, K), lambda i: (0, 0)),
            pl.BlockSpec((1, 1), lambda i: (0, 0)),
            pl.BlockSpec((BN, 1), lambda i: (i, 0)),
        ],
        out_specs=[
            pl.BlockSpec((2, BN, 128), lambda i: (0, i, 0)),
            pl.BlockSpec((BN, K), lambda i: (i, 0)),
            pl.BlockSpec((B, K), lambda i: (0, 0)),
        ],
        out_shape=[
            jax.ShapeDtypeStruct((2, NP, 128), jnp.float32),
            jax.ShapeDtypeStruct((NP, K), jnp.float32),
            jax.ShapeDtypeStruct((B, K), jnp.float32),
        ],
        scratch_shapes=[pltpu.VMEM((B, K), jnp.float32)],
    )(feat2, agg2, wm, bvec, wa, scale, gid2)


def _tc_layer_b_body(featp_ref, sc_ref, smax_ref, gid_ref,
                     ro_ref, den_ref, ro_acc, den_acc):
    i = pl.program_id(0)

    @pl.when(i == 0)
    def _():
        ro_acc[...] = jnp.zeros((B * K, H), jnp.float32)
        den_acc[...] = jnp.zeros((1, B * K), jnp.float32)

    gid = gid_ref[...]
    iota16 = lax.broadcasted_iota(jnp.int32, (BN, B), 1)
    ohf = (gid == iota16).astype(jnp.float32)
    smaxn = jnp.dot(ohf, smax_ref[...], preferred_element_type=jnp.float32)
    ex = jnp.exp(sc_ref[...] - smaxn)
    ext = jnp.concatenate([ex] * B, axis=1)
    bcol = lax.broadcasted_iota(jnp.int32, (BN, B * K), 1) // K
    a = jnp.where(gid == bcol, ext, 0.0)
    fp = jnp.concatenate([featp_ref[0], featp_ref[1]], axis=1)
    ro_acc[...] += lax.dot_general(
        a, fp, (((0,), (0,)), ((), ())), preferred_element_type=jnp.float32)
    den_acc[...] += jnp.sum(a, axis=0, keepdims=True)
    ro_ref[...] = ro_acc[...]
    den_ref[...] = den_acc[...]


def _tc_layer_b(featp2, scores, smax, gid2):
    return pl.pallas_call(
        _tc_layer_b_body,
        grid=(GRID,),
        in_specs=[
            pl.BlockSpec((2, BN, 128), lambda i: (0, i, 0)),
            pl.BlockSpec((BN, K), lambda i: (i, 0)),
            pl.BlockSpec((B, K), lambda i: (0, 0)),
            pl.BlockSpec((BN, 1), lambda i: (i, 0)),
        ],
        out_specs=[
            pl.BlockSpec((B * K, H), lambda i: (0, 0)),
            pl.BlockSpec((1, B * K), lambda i: (0, 0)),
        ],
        out_shape=[
            jax.ShapeDtypeStruct((B * K, H), jnp.float32),
            jax.ShapeDtypeStruct((1, B * K), jnp.float32),
        ],
        scratch_shapes=[
            pltpu.VMEM((B * K, H), jnp.float32),
            pltpu.VMEM((1, B * K), jnp.float32),
        ],
    )(featp2, scores, smax, gid2)


def _tc_head1_body(ro1, dc1, ro2, dc2, ro3, dc3, wct_ref, bc_ref, out_ref):
    def norm(ro, dc):
        d = dc[...]
        d = jnp.where(d > 0, d, 1.0)
        return ro[...] / d

    m = jnp.concatenate([norm(ro1, dc1), norm(ro2, dc2), norm(ro3, dc3)],
                        axis=1)
    ct = jnp.dot(m, wct_ref[...], preferred_element_type=jnp.float32) + bc_ref[...]
    out_ref[...] = jnp.where(ct > 0, ct, 0.01 * ct)


def _tc_head1(ro1, dc1, ro2, dc2, ro3, dc3, wct, bc2):
    return pl.pallas_call(
        _tc_head1_body,
        out_shape=jax.ShapeDtypeStruct((B * K, C), jnp.float32),
    )(ro1, dc1, ro2, dc2, ro3, dc3, wct, bc2)


def _tc_head2_body(fc_ref, wd1_ref, bd1_ref, wd2_ref, bd2_ref, out_ref):
    d1 = jnp.dot(fc_ref[...], wd1_ref[...], preferred_element_type=jnp.float32)
    d1 = d1 + bd1_ref[...]
    d1 = jnp.where(d1 > 0, d1, 0.01 * d1)
    d2 = jnp.dot(d1, wd2_ref[...], preferred_element_type=jnp.float32)
    out_ref[...] = jax.nn.sigmoid(d2 + bd2_ref[...])


def _tc_head2(fc2, wd1q, bd1, wd2, bd2):
    return pl.pallas_call(
        _tc_head2_body,
        out_shape=jax.ShapeDtypeStruct((B, 2), jnp.float32),
    )(fc2, wd1q, bd1, wd2, bd2)


def kernel(x, eps0, eps1, eps2, W1, b1, W2, b2, W3, b3, Wa1, Wa2, Wa3,
           Wc, bc, Wd1, bd1, Wd2, bd2, edge_index, graph_ids):
    f32 = jnp.float32
    xp = jnp.pad(x, ((0, NP - N), (0, 0)))
    gid2 = jnp.pad(graph_ids, (0, NP - N), constant_values=B).reshape(NP, 1)
    src = jnp.pad(edge_index[0], (0, EP - E)).reshape(2560, 128)
    dst = jnp.pad(edge_index[1], (0, EP - E),
                  constant_values=N + 200).reshape(2560, 128)
    src1 = src + NP
    zero128 = jnp.zeros((128, 128), f32)

    wm2 = W2.reshape(2, 128, H)
    wm3 = W3.reshape(2, 128, H)
    s0 = (1.0 + eps0).reshape(1, 1)
    s1 = (1.0 + eps1).reshape(1, 1)
    s2 = (1.0 + eps2).reshape(1, 1)
    wct = Wc.T
    bc2 = bc.reshape(1, C)
    wd1q = Wd1.reshape(C, K, 128).transpose(1, 0, 2).reshape(C * K, 128)
    bd12 = bd1.reshape(1, 128)
    bd22 = bd2.reshape(1, 2)

    agg1 = _make_sc_agg_edgesplit()(xp, src, dst, zero128)
    f1, sc1, smax1 = _tc_layer_a1(xp, agg1.reshape(2, NP, 128), W1,
                                  b1.reshape(1, H), Wa1, s0, gid2)
    ro1, den1 = _tc_layer_b(f1, sc1, smax1, gid2)

    agg2 = _make_sc_agg(128)(f1.reshape(2 * NP, 128), src, src1, dst, zero128)
    f2, sc2, smax2 = _tc_layer_a(f1, agg2.reshape(2, NP, 128), wm2,
                                 b2.reshape(1, H), Wa2, s1, gid2)
    ro2, den2 = _tc_layer_b(f2, sc2, smax2, gid2)

    agg3 = _make_sc_agg(128)(f2.reshape(2 * NP, 128), src, src1, dst, zero128)
    f3, sc3, smax3 = _tc_layer_a(f2, agg3.reshape(2, NP, 128), wm3,
                                 b3.reshape(1, H), Wa3, s2, gid2)
    ro3, den3 = _tc_layer_b(f3, sc3, smax3, gid2)

    conv = _tc_head1(ro1, den1.reshape(B * K, 1), ro2, den2.reshape(B * K, 1),
                     ro3, den3.reshape(B * K, 1), wct, bc2)
    fc2 = conv.reshape(B, C * K)
    return _tc_head2(fc2, wd1q, bd12, Wd2, bd22)


# speedup vs baseline: 4.4347x; 1.0689x over previous
import functools

import jax
import jax.numpy as jnp
from jax import lax
from jax.experimental import pallas as pl
from jax.experimental.pallas import tpu as pltpu
from jax.experimental.pallas import tpu_sc as plsc

N = 10000
NP = 10240
E = 320000
EP = 2560 * 128
B = 16
K = 16
H = 256
C = 64
BN = 1280
GRID = NP // BN
RT = 320
NEG = -1e30


@functools.lru_cache(maxsize=None)
def _make_sc_agg(wh):
    mesh = plsc.VectorSubcoreMesh(
        core_axis_name="c", subcore_axis_name="s", num_cores=2, num_subcores=16
    )

    @functools.partial(
        pl.kernel,
        out_type=jax.ShapeDtypeStruct((2 * NP, wh), jnp.float32),
        mesh=mesh,
        scratch_types=[
            pltpu.VMEM_SHARED((NP, wh), jnp.float32),
            pltpu.VMEM((32, 64), jnp.int32),
            pltpu.VMEM((32, 64), jnp.int32),
            pltpu.VMEM((64, wh), jnp.float32),
            pltpu.VMEM((64, wh), jnp.float32),
            pltpu.VMEM((64, wh), jnp.float32),
            pltpu.VMEM((64, wh), jnp.float32),
            pltpu.VMEM((64, wh), jnp.float32),
            pltpu.SemaphoreType.DMA,
            pltpu.SemaphoreType.DMA,
            pltpu.SemaphoreType.DMA,
            pltpu.SemaphoreType.DMA,
        ],
    )
    def sc_agg(feat_hbm, src0_hbm, src1_hbm, dst_hbm, zero_hbm, out_hbm,
               acc, src_c, dst_c, wbuf, rb0, rb1, rb2, rb3,
               sem0, sem1, sem2, sem3):
        c = lax.axis_index("c")
        s = lax.axis_index("s")

        pltpu.sync_copy(zero_hbm, wbuf)
        for i in range(10):
            pltpu.sync_copy(wbuf, acc.at[pl.ds(s * 640 + i * 64, 64)])
        plsc.subcore_barrier()

        tile_row0 = s * RT

        def chunk(ck, carry):
            row0 = tile_row0 + ck * 32

            @pl.when(c == 0)
            def _():
                pltpu.sync_copy(src0_hbm.at[pl.ds(row0, 32)], src_c)

            @pl.when(c == 1)
            def _():
                pltpu.sync_copy(src1_hbm.at[pl.ds(row0, 32)], src_c)

            pltpu.sync_copy(dst_hbm.at[pl.ds(row0, 32)], dst_c)
            bufs = (rb0, rb1, rb2, rb3)
            sems = (sem0, sem1, sem2, sem3)
            descs = [None] * 4
            for jj in range(4):
                descs[jj] = pltpu.async_copy(
                    feat_hbm.at[src_c.at[jj]], bufs[jj], sems[jj])
            for jj in range(32):
                p = jj % 4
                descs[p].wait()
                pltpu.sync_copy(bufs[p], acc.at[dst_c.at[jj]], add=True)
                if jj + 4 < 32:
                    descs[p] = pltpu.async_copy(
                        feat_hbm.at[src_c.at[jj + 4]], bufs[p], sems[p])
            return carry

        lax.fori_loop(0, RT // 32, chunk, 0)
        plsc.subcore_barrier()

        for i in range(10):
            r0 = s * 640 + i * 64
            pltpu.sync_copy(acc.at[pl.ds(r0, 64)], wbuf)
            pltpu.sync_copy(wbuf, out_hbm.at[pl.ds(c * NP + r0, 64)])

    return sc_agg


@functools.lru_cache(maxsize=None)
def _make_sc_agg_edgesplit():
    mesh = plsc.VectorSubcoreMesh(
        core_axis_name="c", subcore_axis_name="s", num_cores=2, num_subcores=16
    )
    rt = 160

    @functools.partial(
        pl.kernel,
        out_type=jax.ShapeDtypeStruct((2 * NP, 128), jnp.float32),
        mesh=mesh,
        scratch_types=[
            pltpu.VMEM_SHARED((NP, 128), jnp.float32),
            pltpu.VMEM((32, 64), jnp.int32),
            pltpu.VMEM((32, 64), jnp.int32),
            pltpu.VMEM((64, 128), jnp.float32),
            pltpu.VMEM((64, 128), jnp.float32),
            pltpu.VMEM((64, 128), jnp.float32),
            pltpu.VMEM((64, 128), jnp.float32),
            pltpu.VMEM((64, 128), jnp.float32),
            pltpu.SemaphoreType.DMA,
            pltpu.SemaphoreType.DMA,
            pltpu.SemaphoreType.DMA,
            pltpu.SemaphoreType.DMA,
        ],
    )
    def sc_agg1(feat_hbm, src_hbm, dst_hbm, zero_hbm, out_hbm,
                acc, src_c, dst_c, wbuf, rb0, rb1, rb2, rb3,
                sem0, sem1, sem2, sem3):
        c = lax.axis_index("c")
        s = lax.axis_index("s")

        pltpu.sync_copy(zero_hbm, wbuf)
        for i in range(10):
            pltpu.sync_copy(wbuf, acc.at[pl.ds(s * 640 + i * 64, 64)])
        plsc.subcore_barrier()

        tile_row0 = (c * 16 + s) * rt

        def chunk(ck, carry):
            row0 = tile_row0 + ck * 32
            pltpu.sync_copy(src_hbm.at[pl.ds(row0, 32)], src_c)
            pltpu.sync_copy(dst_hbm.at[pl.ds(row0, 32)], dst_c)
            bufs = (rb0, rb1, rb2, rb3)
            sems = (sem0, sem1, sem2, sem3)
            descs = [None] * 4
            for jj in range(4):
                descs[jj] = pltpu.async_copy(
                    feat_hbm.at[src_c.at[jj]], bufs[jj], sems[jj])
            for jj in range(32):
                p = jj % 4
                descs[p].wait()
                pltpu.sync_copy(bufs[p], acc.at[dst_c.at[jj]], add=True)
                if jj + 4 < 32:
                    descs[p] = pltpu.async_copy(
                        feat_hbm.at[src_c.at[jj + 4]], bufs[p], sems[p])
            return carry

        lax.fori_loop(0, rt // 32, chunk, 0)
        plsc.subcore_barrier()

        for i in range(10):
            r0 = s * 640 + i * 64
            pltpu.sync_copy(acc.at[pl.ds(r0, 64)], wbuf)
            pltpu.sync_copy(wbuf, out_hbm.at[pl.ds(c * NP + r0, 64)])

    return sc_agg1


def _finish_layer_a(z, wa_ref, gid_ref, featp_ref, scores_ref, smax_ref,
                    smax_acc):
    i = pl.program_id(0)
    fp = jnp.where(z > 0, z, 0.01 * z)
    featp_ref[0] = fp[:, :128]
    featp_ref[1] = fp[:, 128:]
    sc_blk = jnp.dot(fp, wa_ref[...], preferred_element_type=jnp.float32)
    scores_ref[...] = sc_blk

    @pl.when(i == 0)
    def _():
        smax_acc[...] = jnp.full((B, K), NEG, jnp.float32)

    gid = gid_ref[...]
    rows = []
    for b in range(B):
        mb = jnp.where(gid == b, sc_blk, NEG)
        rows.append(jnp.max(mb, axis=0, keepdims=True))
    smax_acc[...] = jnp.maximum(smax_acc[...], jnp.concatenate(rows, axis=0))
    smax_ref[...] = smax_acc[...]


def _tc_layer_a_body(feat_ref, agg_ref, wm_ref, b_ref, wa_ref, s_ref, gid_ref,
                     featp_ref, scores_ref, smax_ref, smax_acc):
    sc = s_ref[0, 0]
    h0 = feat_ref[0] * sc + agg_ref[0]
    h1 = feat_ref[1] * sc + agg_ref[1]
    z = (jnp.dot(h0, wm_ref[0], preferred_element_type=jnp.float32)
         + jnp.dot(h1, wm_ref[1], preferred_element_type=jnp.float32)
         + b_ref[...])
    _finish_layer_a(z, wa_ref, gid_ref, featp_ref, scores_ref, smax_ref,
                    smax_acc)


def _tc_layer_a1_body(feat_ref, agg_ref, w_ref, b_ref, wa_ref, s_ref, gid_ref,
                      featp_ref, scores_ref, smax_ref, smax_acc):
    sc = s_ref[0, 0]
    h = feat_ref[...] * sc + agg_ref[0] + agg_ref[1]
    z = jnp.dot(h, w_ref[...], preferred_element_type=jnp.float32) + b_ref[...]
    _finish_layer_a(z, wa_ref, gid_ref, featp_ref, scores_ref, smax_ref,
                    smax_acc)


def _tc_layer_a1(xp, aggp, w1, bvec, wa, scale, gid2):
    return pl.pallas_call(
        _tc_layer_a1_body,
        grid=(GRID,),
        in_specs=[
            pl.BlockSpec((BN, 128), lambda i: (i, 0)),
            pl.BlockSpec((2, BN, 128), lambda i: (0, i, 0)),
            pl.BlockSpec((128, H), lambda i: (0, 0)),
            pl.BlockSpec((1, H), lambda i: (0, 0)),
            pl.BlockSpec((H, K), lambda i: (0, 0)),
            pl.BlockSpec((1, 1), lambda i: (0, 0)),
            pl.BlockSpec((BN, 1), lambda i: (i, 0)),
        ],
        out_specs=[
            pl.BlockSpec((2, BN, 128), lambda i: (0, i, 0)),
            pl.BlockSpec((BN, K), lambda i: (i, 0)),
            pl.BlockSpec((B, K), lambda i: (0, 0)),
        ],
        out_shape=[
            jax.ShapeDtypeStruct((2, NP, 128), jnp.float32),
            jax.ShapeDtypeStruct((NP, K), jnp.float32),
            jax.ShapeDtypeStruct((B, K), jnp.float32),
        ],
        scratch_shapes=[pltpu.VMEM((B, K), jnp.float32)],
    )(xp, aggp, w1, bvec, wa, scale, gid2)


def _tc_layer_a(feat2, agg2, wm, bvec, wa, scale, gid2):
    whin = feat2.shape[2]
    return pl.pallas_call(
        _tc_layer_a_body,
        grid=(GRID,),
        in_specs=[
            pl.BlockSpec((2, BN, whin), lambda i: (0, i, 0)),
            pl.BlockSpec((2, BN, whin), lambda i: (0, i, 0)),
            pl.BlockSpec((2, whin, H), lambda i: (0, 0, 0)),
            pl.BlockSpec((1, H), lambda i: (0, 0)),
            pl.BlockSpec((H, K), lambda i: (0, 0)),
            pl.BlockSpec((1, 1), lambda i: (0, 0)),
            pl.BlockSpec((BN, 1), lambda i: (i, 0)),
        ],
        out_specs=[
            pl.BlockSpec((2, BN, 128), lambda i: (0, i, 0)),
            pl.BlockSpec((BN, K), lambda i: (i, 0)),
            pl.BlockSpec((B, K), lambda i: (0, 0)),
        ],
        out_shape=[
            jax.ShapeDtypeStruct((2, NP, 128), jnp.float32),
            jax.ShapeDtypeStruct((NP, K), jnp.float32),
            jax.ShapeDtypeStruct((B, K), jnp.float32),
        ],
        scratch_shapes=[pltpu.VMEM((B, K), jnp.float32)],
    )(feat2, agg2, wm, bvec, wa, scale, gid2)


def _tc_layer_b_body(featp_ref, sc_ref, smax_ref, gid_ref,
                     ro_ref, den_ref, ro_acc, den_acc):
    i = pl.program_id(0)

    @pl.when(i == 0)
    def _():
        ro_acc[...] = jnp.zeros((B * K, H), jnp.float32)
        den_acc[...] = jnp.zeros((1, B * K), jnp.float32)

    gid = gid_ref[...]
    iota16 = lax.broadcasted_iota(jnp.int32, (BN, B), 1)
    ohf = (gid == iota16).astype(jnp.float32)
    smaxn = jnp.dot(ohf, smax_ref[...], preferred_element_type=jnp.float32)
    ex = jnp.exp(sc_ref[...] - smaxn)
    ext = jnp.concatenate([ex] * B, axis=1)
    bcol = lax.broadcasted_iota(jnp.int32, (BN, B * K), 1) // K
    a = jnp.where(gid == bcol, ext, 0.0)
    fp = jnp.concatenate([featp_ref[0], featp_ref[1]], axis=1)
    ro_acc[...] += lax.dot_general(
        a, fp, (((0,), (0,)), ((), ())), preferred_element_type=jnp.float32)
    den_acc[...] += jnp.sum(a, axis=0, keepdims=True)
    ro_ref[...] = ro_acc[...]
    den_ref[...] = den_acc[...]


def _tc_layer_b(featp2, scores, smax, gid2):
    return pl.pallas_call(
        _tc_layer_b_body,
        grid=(GRID,),
        in_specs=[
            pl.BlockSpec((2, BN, 128), lambda i: (0, i, 0)),
            pl.BlockSpec((BN, K), lambda i: (i, 0)),
            pl.BlockSpec((B, K), lambda i: (0, 0)),
            pl.BlockSpec((BN, 1), lambda i: (i, 0)),
        ],
        out_specs=[
            pl.BlockSpec((B * K, H), lambda i: (0, 0)),
            pl.BlockSpec((1, B * K), lambda i: (0, 0)),
        ],
        out_shape=[
            jax.ShapeDtypeStruct((B * K, H), jnp.float32),
            jax.ShapeDtypeStruct((1, B * K), jnp.float32),
        ],
        scratch_shapes=[
            pltpu.VMEM((B * K, H), jnp.float32),
            pltpu.VMEM((1, B * K), jnp.float32),
        ],
    )(featp2, scores, smax, gid2)


def _tc_head1_body(ro1, dc1, ro2, dc2, ro3, dc3, wct_ref, bc_ref, out_ref):
    def norm(ro, dc):
        d = dc[...]
        d = jnp.where(d > 0, d, 1.0)
        return ro[...] / d

    m = jnp.concatenate([norm(ro1, dc1), norm(ro2, dc2), norm(ro3, dc3)],
                        axis=1)
    ct = jnp.dot(m, wct_ref[...], preferred_element_type=jnp.float32) + bc_ref[...]
    out_ref[...] = jnp.where(ct > 0, ct, 0.01 * ct)


def _tc_head1(ro1, dc1, ro2, dc2, ro3, dc3, wct, bc2):
    return pl.pallas_call(
        _tc_head1_body,
        out_shape=jax.ShapeDtypeStruct((B * K, C), jnp.float32),
    )(ro1, dc1, ro2, dc2, ro3, dc3, wct, bc2)


def _tc_head2_body(fc_ref, wd1_ref, bd1_ref, wd2_ref, bd2_ref, out_ref):
    d1 = jnp.dot(fc_ref[...], wd1_ref[...], preferred_element_type=jnp.float32)
    d1 = d1 + bd1_ref[...]
    d1 = jnp.where(d1 > 0, d1, 0.01 * d1)
    d2 = jnp.dot(d1, wd2_ref[...], preferred_element_type=jnp.float32)
    out_ref[...] = jax.nn.sigmoid(d2 + bd2_ref[...])


def _tc_head2(fc2, wd1q, bd1, wd2, bd2):
    return pl.pallas_call(
        _tc_head2_body,
        out_shape=jax.ShapeDtypeStruct((B, 2), jnp.float32),
    )(fc2, wd1q, bd1, wd2, bd2)


def kernel(x, eps0, eps1, eps2, W1, b1, W2, b2, W3, b3, Wa1, Wa2, Wa3,
           Wc, bc, Wd1, bd1, Wd2, bd2, edge_index, graph_ids):
    f32 = jnp.float32
    xp = jnp.pad(x, ((0, NP - N), (0, 0)))
    gid2 = jnp.pad(graph_ids, (0, NP - N), constant_values=B).reshape(NP, 1)
    src = jnp.pad(edge_index[0], (0, EP - E)).reshape(5120, 64)
    dst = jnp.pad(edge_index[1], (0, EP - E),
                  constant_values=N + 200).reshape(5120, 64)
    src1 = src + NP
    zero128 = jnp.zeros((64, 128), f32)

    wm2 = W2.reshape(2, 128, H)
    wm3 = W3.reshape(2, 128, H)
    s0 = (1.0 + eps0).reshape(1, 1)
    s1 = (1.0 + eps1).reshape(1, 1)
    s2 = (1.0 + eps2).reshape(1, 1)
    wct = Wc.T
    bc2 = bc.reshape(1, C)
    wd1q = Wd1.reshape(C, K, 128).transpose(1, 0, 2).reshape(C * K, 128)
    bd12 = bd1.reshape(1, 128)
    bd22 = bd2.reshape(1, 2)

    agg1 = _make_sc_agg_edgesplit()(xp, src, dst, zero128)
    f1, sc1, smax1 = _tc_layer_a1(xp, agg1.reshape(2, NP, 128), W1,
                                  b1.reshape(1, H), Wa1, s0, gid2)
    ro1, den1 = _tc_layer_b(f1, sc1, smax1, gid2)

    agg2 = _make_sc_agg(128)(f1.reshape(2 * NP, 128), src, src1, dst, zero128)
    f2, sc2, smax2 = _tc_layer_a(f1, agg2.reshape(2, NP, 128), wm2,
                                 b2.reshape(1, H), Wa2, s1, gid2)
    ro2, den2 = _tc_layer_b(f2, sc2, smax2, gid2)

    agg3 = _make_sc_agg(128)(f2.reshape(2 * NP, 128), src, src1, dst, zero128)
    f3, sc3, smax3 = _tc_layer_a(f2, agg3.reshape(2, NP, 128), wm3,
                                 b3.reshape(1, H), Wa3, s2, gid2)
    ro3, den3 = _tc_layer_b(f3, sc3, smax3, gid2)

    conv = _tc_head1(ro1, den1.reshape(B * K, 1), ro2, den2.reshape(B * K, 1),
                     ro3, den3.reshape(B * K, 1), wct, bc2)
    fc2 = conv.reshape(B, C * K)
    return _tc_head2(fc2, wd1q, bd12, Wd2, bd22)
